# baseline (speedup 1.0000x reference)
import functools
import math

import jax
import jax.numpy as jnp
from jax import lax
from jax.experimental import pallas as pl
from jax.experimental.pallas import tpu as pltpu
from jax.experimental.pallas import tpu_sc as plsc

N_ROWS = 262144
B_OBS = 16384
IN_SZ = 16
HID = 64
PREP = 8

NC, NS, L = 2, 16, 16
NW = NC * NS
BPW = B_OBS // NW

_LOGC = math.log(math.sqrt(2.0 * math.pi))


def _sc_gather_body(h_hbm, p_hbm, idx_hbm, hob_hbm, pob_hbm,
                    idx_v, hbuf, pbuf, sem_h, sem_p):
    wid = lax.axis_index("s") * NC + lax.axis_index("c")
    base = wid * BPW
    pltpu.sync_copy(idx_hbm.at[pl.ds(wid * 4, 4)], idx_v)
    cps = []
    for j in range(4):
        cps.append(pltpu.async_copy(h_hbm.at[idx_v.at[j]],
                                    hbuf.at[pl.ds(j * 128, 128)], sem_h))
        cps.append(pltpu.async_copy(p_hbm.at[idx_v.at[j]],
                                    pbuf.at[pl.ds(j * 128, 128)], sem_p))
    for c in cps:
        c.wait()
    pltpu.sync_copy(hbuf, hob_hbm.at[pl.ds(base, BPW)])
    pltpu.sync_copy(pbuf, pob_hbm.at[pl.ds(base, BPW)])


_sc_gather = functools.partial(
    pl.kernel,
    _sc_gather_body,
    out_type=(jax.ShapeDtypeStruct((B_OBS, HID), jnp.float32),
              jax.ShapeDtypeStruct((B_OBS, 2 * IN_SZ), jnp.float32)),
    mesh=plsc.VectorSubcoreMesh(core_axis_name="c", subcore_axis_name="s"),
    compiler_params=pltpu.CompilerParams(use_tc_tiling_on_sc=False),
    scratch_types=[pltpu.VMEM((4, 128), jnp.int32),
                   pltpu.VMEM((BPW, HID), jnp.float32),
                   pltpu.VMEM((BPW, 2 * IN_SZ), jnp.float32),
                   pltpu.SemaphoreType.DMA,
                   pltpu.SemaphoreType.DMA],
)


def _prep_body(x_ref, m_ref, p_ref, wbig_ref, bbig_ref, losses_ref, c_ref):
    x = x_ref[...]
    m = m_ref[...]
    pob = p_ref[...]
    mean = pob[:, :IN_SZ]
    logvar = pob[:, IN_SZ:]
    err = (x - mean) * jnp.exp(-0.5 * logvar)
    losses_ref[...] = 0.5 * ((err * err + logvar + 2.0 * _LOGC) * m)
    stack = jnp.concatenate([x, mean, logvar, err], axis=1)
    c = jnp.dot(stack, wbig_ref[...], preferred_element_type=jnp.float32)
    c = jnp.maximum(c + bbig_ref[...], 0.0)
    r = m.shape[0]
    m_rep = jnp.broadcast_to(m[:, :, None], (r, IN_SZ, PREP)).reshape(r, IN_SZ * PREP)
    c_ref[...] = c * m_rep


def _gru_body(xin_ref, hob_ref, gk_ref, grk_ref, gib_ref, grb_ref, hnew_ref):
    x = xin_ref[...]
    h0 = hob_ref[...]
    mx = jnp.dot(x, gk_ref[...], preferred_element_type=jnp.float32) + gib_ref[...]
    mi = jnp.dot(h0, grk_ref[...], preferred_element_type=jnp.float32) + grb_ref[...]
    z = jax.nn.sigmoid(mx[:, :HID] + mi[:, :HID])
    r = jax.nn.sigmoid(mx[:, HID:2 * HID] + mi[:, HID:2 * HID])
    hh = jnp.tanh(mx[:, 2 * HID:] + r * mi[:, 2 * HID:])
    hnew_ref[...] = z * h0 + (1.0 - z) * hh


def kernel(h, p, X_obs, M_obs, i_obs, w_prep, bias_prep, gru_kernel,
           gru_rec_kernel, gru_input_bias, gru_rec_bias):
    eye = jnp.eye(IN_SZ, dtype=jnp.float32)
    W_big = (jnp.transpose(w_prep, (1, 0, 2))[:, :, None, :]
             * eye[None, :, :, None]).reshape(4 * IN_SZ, IN_SZ * PREP)
    bias_big = bias_prep.reshape(1, IN_SZ * PREP)

    idx2d = i_obs.reshape(B_OBS // 128, 128)
    h_obs, p_obs = _sc_gather()(h, p, idx2d)

    R = 2048
    grid = (B_OBS // R,)
    losses, c = pl.pallas_call(
        _prep_body,
        grid=grid,
        in_specs=[pl.BlockSpec((R, IN_SZ), lambda i: (i, 0)),
                  pl.BlockSpec((R, IN_SZ), lambda i: (i, 0)),
                  pl.BlockSpec((R, 2 * IN_SZ), lambda i: (i, 0)),
                  pl.BlockSpec((4 * IN_SZ, IN_SZ * PREP), lambda i: (0, 0)),
                  pl.BlockSpec((1, IN_SZ * PREP), lambda i: (0, 0))],
        out_specs=[pl.BlockSpec((R, IN_SZ), lambda i: (i, 0)),
                   pl.BlockSpec((R, IN_SZ * PREP), lambda i: (i, 0))],
        out_shape=[jax.ShapeDtypeStruct((B_OBS, IN_SZ), jnp.float32),
                   jax.ShapeDtypeStruct((B_OBS, IN_SZ * PREP), jnp.float32)],
    )(X_obs, M_obs, p_obs, W_big, bias_big)

    gru_in = (c.reshape(B_OBS, IN_SZ, PREP)
                .transpose(2, 0, 1)
                .reshape(B_OBS, IN_SZ * PREP))

    h_new = pl.pallas_call(
        _gru_body,
        grid=grid,
        in_specs=[pl.BlockSpec((R, IN_SZ * PREP), lambda i: (i, 0)),
                  pl.BlockSpec((R, HID), lambda i: (i, 0)),
                  pl.BlockSpec((IN_SZ * PREP, 3 * HID), lambda i: (0, 0)),
                  pl.BlockSpec((HID, 3 * HID), lambda i: (0, 0)),
                  pl.BlockSpec((1, 3 * HID), lambda i: (0, 0)),
                  pl.BlockSpec((1, 3 * HID), lambda i: (0, 0))],
        out_specs=pl.BlockSpec((R, HID), lambda i: (i, 0)),
        out_shape=jax.ShapeDtypeStruct((B_OBS, HID), jnp.float32),
    )(gru_in, h_obs, gru_kernel, gru_rec_kernel,
      gru_input_bias.reshape(1, 3 * HID), gru_rec_bias.reshape(1, 3 * HID))

    h_out = h.at[i_obs].set(h_new)
    return (h_out, losses)
